# baseline (speedup 1.0000x reference)
import functools

import jax
import jax.numpy as jnp
from jax import lax
from jax.experimental import pallas as pl
from jax.experimental.pallas import tpu as pltpu
from jax.experimental.pallas import tpu_sc as plsc

_BATCH = 16384
_HIST = 200
_N = _BATCH * _HIST
_NC = 2
_NS = 16
_NW = _NC * _NS
_PER_W = _N // _NW
_CHUNK = 12800
_NCHUNK = _PER_W // _CHUNK


_TBL = 1000000
_TBL_SLICE = _TBL // 10
_STAGE_SIZES = [10000] * 10
_STAGE_OFFS = [10000 * j for j in range(10)]


def _sc_gather(table, idx):
    mesh = plsc.VectorSubcoreMesh(core_axis_name="c", subcore_axis_name="s")

    @functools.partial(
        pl.kernel,
        mesh=mesh,
        out_type=jax.ShapeDtypeStruct((_N,), jnp.float32),
        scratch_types=[
            pltpu.VMEM_SHARED((_TBL,), jnp.float32),
            pltpu.VMEM((_CHUNK,), jnp.int32),
            pltpu.VMEM((_CHUNK,), jnp.int32),
            pltpu.VMEM((_CHUNK,), jnp.float32),
            pltpu.VMEM((_CHUNK,), jnp.float32),
            pltpu.SemaphoreType.DMA,
            pltpu.SemaphoreType.DMA,
            pltpu.SemaphoreType.DMA,
            pltpu.SemaphoreType.DMA,
            pltpu.SemaphoreType.DMA,
            pltpu.SemaphoreType.DMA,
        ],
    )
    def k(table_hbm, idx_hbm, out_hbm,
          tbl_s, idx0, idx1, rows0, rows1,
          si0, si1, sg0, sg1, ss0, ss1):
        sid = lax.axis_index("s")
        wid = sid * _NC + lax.axis_index("c")
        base = wid * _PER_W

        pre0 = pltpu.async_copy(
            idx_hbm.at[pl.ds(base, _CHUNK)], idx0, si0)
        pre1 = pltpu.async_copy(
            idx_hbm.at[pl.ds(base + _CHUNK, _CHUNK)], idx1, si1)

        @pl.when(sid < 10)
        def _stage():
            toff = pl.multiple_of(sid * _TBL_SLICE, 8)
            bufs = (rows0, rows1)
            ld = (sg0, sg1)
            st = (ss0, ss1)
            nst = len(_STAGE_SIZES)
            loads = [None] * nst
            stores = [None] * nst
            loads[0] = pltpu.async_copy(
                table_hbm.at[pl.ds(toff, _STAGE_SIZES[0])],
                bufs[0].at[pl.ds(0, _STAGE_SIZES[0])], ld[0])
            for j in range(nst):
                b = j % 2
                o = toff + _STAGE_OFFS[j]
                loads[j].wait()
                if j >= 1:
                    stores[j - 1].wait()
                if j + 1 < nst:
                    loads[j + 1] = pltpu.async_copy(
                        table_hbm.at[pl.ds(toff + _STAGE_OFFS[j + 1],
                                           _STAGE_SIZES[j + 1])],
                        bufs[1 - b].at[pl.ds(0, _STAGE_SIZES[j + 1])],
                        ld[1 - b])
                stores[j] = pltpu.async_copy(
                    bufs[b].at[pl.ds(0, _STAGE_SIZES[j])],
                    tbl_s.at[pl.ds(o, _STAGE_SIZES[j])], st[b])
            stores[nst - 1].wait()

        plsc.subcore_barrier()
        idx_v = (idx0, idx1)
        rows_v = (rows0, rows1)
        s_idx = (si0, si1)
        s_gat = (sg0, sg1)
        s_out = (ss0, ss1)

        idx_loads = [None] * _NCHUNK
        stores = [None] * _NCHUNK

        idx_loads[0] = pre0
        idx_loads[1] = pre1
        for c in range(_NCHUNK):
            b = c % 2
            off = base + c * _CHUNK
            idx_loads[c].wait()
            if c >= 2:
                stores[c - 2].wait()
            pltpu.async_copy(tbl_s.at[idx_v[b]], rows_v[b],
                             s_gat[b]).wait()
            if c + 2 < _NCHUNK:
                idx_loads[c + 2] = pltpu.async_copy(
                    idx_hbm.at[pl.ds(off + 2 * _CHUNK, _CHUNK)],
                    idx_v[b], s_idx[b])
            stores[c] = pltpu.async_copy(
                rows_v[b], out_hbm.at[pl.ds(off, _CHUNK)], s_out[b])
        stores[_NCHUNK - 2].wait()
        stores[_NCHUNK - 1].wait()

    return k(table, idx)


def kernel(input_ids, witness_ids, W):
    idx = witness_ids.T.reshape(_N)
    table = W[jnp.arange(_TBL, dtype=jnp.int32), 0]
    out = _sc_gather(table, idx)
    return out.reshape(_HIST, _BATCH, 1).transpose((1, 0, 2))


# speedup vs baseline: 1.4520x; 1.4520x over previous
import functools

import jax
import jax.numpy as jnp
from jax import lax
from jax.experimental import pallas as pl
from jax.experimental.pallas import tpu as pltpu
from jax.experimental.pallas import tpu_sc as plsc

_BATCH = 16384
_HIST = 200
_N = _BATCH * _HIST
_NC = 2
_NS = 16
_NW = _NC * _NS
_PER_W = _N // _NW
_CHUNK = 12800
_NCHUNK = _PER_W // _CHUNK


_TBL = 1000000
_TBL_SLICE = _TBL // 10
_STAGE_SIZES = [10000] * 10
_STAGE_OFFS = [10000 * j for j in range(10)]


def _sc_gather(table, idx):
    mesh = plsc.VectorSubcoreMesh(core_axis_name="c", subcore_axis_name="s")

    @functools.partial(
        pl.kernel,
        mesh=mesh,
        out_type=jax.ShapeDtypeStruct((_N,), jnp.float32),
        scratch_types=[
            pltpu.VMEM_SHARED((_TBL,), jnp.float32),
            pltpu.VMEM((_CHUNK,), jnp.int32),
            pltpu.VMEM((_CHUNK,), jnp.int32),
            pltpu.VMEM((_CHUNK,), jnp.float32),
            pltpu.VMEM((_CHUNK,), jnp.float32),
            pltpu.SemaphoreType.DMA,
            pltpu.SemaphoreType.DMA,
            pltpu.SemaphoreType.DMA,
            pltpu.SemaphoreType.DMA,
            pltpu.SemaphoreType.DMA,
            pltpu.SemaphoreType.DMA,
        ],
    )
    def k(table_hbm, idx_hbm, out_hbm,
          tbl_s, idx0, idx1, rows0, rows1,
          si0, si1, sg0, sg1, ss0, ss1):
        sid = lax.axis_index("s")
        wid = sid * _NC + lax.axis_index("c")
        base = wid * _PER_W

        pre0 = pltpu.async_copy(
            idx_hbm.at[pl.ds(base, _CHUNK)], idx0, si0)
        pre1 = pltpu.async_copy(
            idx_hbm.at[pl.ds(base + _CHUNK, _CHUNK)], idx1, si1)

        @pl.when(sid < 10)
        def _stage():
            toff = pl.multiple_of(sid * _TBL_SLICE, 8)
            bufs = (rows0, rows1)
            ld = (sg0, sg1)
            st = (ss0, ss1)
            nst = len(_STAGE_SIZES)
            loads = [None] * nst
            stores = [None] * nst
            loads[0] = pltpu.async_copy(
                table_hbm.at[pl.ds(toff, _STAGE_SIZES[0])],
                bufs[0].at[pl.ds(0, _STAGE_SIZES[0])], ld[0])
            for j in range(nst):
                b = j % 2
                o = toff + _STAGE_OFFS[j]
                loads[j].wait()
                if j >= 1:
                    stores[j - 1].wait()
                if j + 1 < nst:
                    loads[j + 1] = pltpu.async_copy(
                        table_hbm.at[pl.ds(toff + _STAGE_OFFS[j + 1],
                                           _STAGE_SIZES[j + 1])],
                        bufs[1 - b].at[pl.ds(0, _STAGE_SIZES[j + 1])],
                        ld[1 - b])
                stores[j] = pltpu.async_copy(
                    bufs[b].at[pl.ds(0, _STAGE_SIZES[j])],
                    tbl_s.at[pl.ds(o, _STAGE_SIZES[j])], st[b])
            stores[nst - 1].wait()

        plsc.subcore_barrier()
        idx_v = (idx0, idx1)
        rows_v = (rows0, rows1)
        s_idx = (si0, si1)
        s_gat = (sg0, sg1)
        s_out = (ss0, ss1)

        idx_loads = [None] * _NCHUNK
        stores = [None] * _NCHUNK

        idx_loads[0] = pre0
        idx_loads[1] = pre1
        for c in range(_NCHUNK):
            b = c % 2
            off = base + c * _CHUNK
            idx_loads[c].wait()
            if c >= 2:
                stores[c - 2].wait()
            pltpu.async_copy(tbl_s.at[idx_v[b]], rows_v[b],
                             s_gat[b]).wait()
            if c + 2 < _NCHUNK:
                idx_loads[c + 2] = pltpu.async_copy(
                    idx_hbm.at[pl.ds(off + 2 * _CHUNK, _CHUNK)],
                    idx_v[b], s_idx[b])
            stores[c] = pltpu.async_copy(
                rows_v[b], out_hbm.at[pl.ds(off, _CHUNK)], s_out[b])
        stores[_NCHUNK - 2].wait()
        stores[_NCHUNK - 1].wait()

    return k(table, idx)


def kernel(input_ids, witness_ids, W):
    idx = witness_ids.T.reshape(_N)
    table = W.T.reshape(_TBL)
    out = _sc_gather(table, idx)
    return out.reshape(_HIST, _BATCH, 1).transpose((1, 0, 2))
